# baseline (speedup 1.0000x reference)
import jax
import jax.numpy as jnp
import numpy as np
from jax.experimental import pallas as pl
from jax.experimental.pallas import tpu as pltpu

B = 4
N = 512
HIDDEN = 64
NUM_RADIAL = 8
MAX_ANG = 4
CUTOFF = 5.0
TWO_H = 2 * HIDDEN

TI = 64


def _body(hi_ref, hall_ref, posi_ref, posallT_ref,
          w1a_ref, w1b_ref, wf_ref, bpre_ref, w2_ref, b2_ref,
          w3a_ref, w3b_ref, b3_ref, w4_ref, b4_ref, g_ref, be_ref,
          o_ref):
    hi = hi_ref[0]
    hall = hall_ref[0]
    pi = posi_ref[0]
    pT = posallT_ref[0]

    pxi = pi[:, 0:1]
    pyi = pi[:, 1:2]
    pxj = pT[0:1, :]
    pyj = pT[1:2, :]

    pdx = pxi - pxj
    pdy = pyi - pyj
    d2 = pdx * pdx + pdy * pdy
    dist = jnp.sqrt(jnp.maximum(d2, 1e-16))
    inv = 1.0 / (dist + 1e-8)
    dx = pdx * inv
    dy = pdy * inv
    denom = dx * dx + dy * dy
    msk = denom < 1e-12
    xs = jnp.where(msk, 1.0, dx)
    ys = jnp.where(msk, 0.0, dy)
    rinv = jax.lax.rsqrt(jnp.where(msk, 1.0, denom))
    c1 = xs * rinv
    s1 = ys * rinv

    xq = dist * (1.0 / CUTOFF)
    x2 = xq * xq
    x6 = x2 * x2 * x2
    t1 = 1.0 - x6
    t2 = t1 * t1
    cut = jnp.where(xq < 1.0, t2 * t2 * t2, 0.0)
    csum = jnp.sum(cut, axis=1, keepdims=True)

    width_inv = NUM_RADIAL / CUTOFF
    dw = dist * width_inv
    cstep = NUM_RADIAL / (NUM_RADIAL - 1.0)
    planes = []
    for k in range(NUM_RADIAL):
        t = dw - (k * cstep)
        planes.append(jnp.exp(-(t * t)))
    cc2 = c1 * c1
    cc3 = cc2 * c1
    cc4 = cc2 * cc2
    planes += [c1, cc2, cc3, cc4, s1, s1 * c1, s1 * cc2, s1 * cc3]
    G = jnp.stack(planes, axis=0)

    rowA = (jnp.dot(hi, w1a_ref[...], preferred_element_type=jnp.float32)
            + bpre_ref[...])
    colB = jnp.dot(hall, w1b_ref[...], preferred_element_type=jnp.float32)
    pref = jax.lax.dot_general(
        G.astype(jnp.bfloat16), wf_ref[...], (((0,), (0,)), ((), ())),
        preferred_element_type=jnp.float32)
    pre = pref + rowA[:, None, :] + colB[None, :, :]
    hid = pre * (0.5 * jnp.tanh(0.5 * pre) + 0.5)

    ch = jax.lax.dot_general(
        cut, hid, (((1,), (1,)), ((0,), (0,))),
        preferred_element_type=jnp.float32)
    agg = (jnp.dot(ch, w2_ref[...], preferred_element_type=jnp.float32)
           + csum * b2_ref[...])

    ui = (jnp.dot(hi, w3a_ref[...], preferred_element_type=jnp.float32)
          + jnp.dot(agg, w3b_ref[...], preferred_element_type=jnp.float32)
          + b3_ref[...])
    u = ui * jax.nn.sigmoid(ui)
    u2 = jnp.dot(u, w4_ref[...], preferred_element_type=jnp.float32) + b4_ref[...]
    y = hi + u2
    mu = jnp.mean(y, axis=-1, keepdims=True)
    r = y - mu
    var = jnp.mean(r * r, axis=-1, keepdims=True)
    o_ref[0] = (r * jax.lax.rsqrt(var + 1e-5)) * g_ref[...] + be_ref[...]


@jax.jit
def kernel(h, pos, W1, b1, W2, b2, W3, b3, W4, b4, gamma, beta):
    W1a = W1[:HIDDEN]
    W1b = W1[HIDDEN:TWO_H]
    W1r = W1[TWO_H:TWO_H + NUM_RADIAL]
    Wang = W1[TWO_H + NUM_RADIAL:]
    w0, wc1, ws1, wc2, ws2, wc3, ws3, wc4, ws4 = [Wang[i] for i in range(9)]
    WF = jnp.concatenate([
        W1r,
        jnp.stack([
            wc1 - 3.0 * wc3,
            2.0 * wc2 - 8.0 * wc4,
            4.0 * wc3,
            8.0 * wc4,
            ws1 - ws3,
            2.0 * ws2 - 4.0 * ws4,
            4.0 * ws3,
            8.0 * ws4,
        ]),
    ], axis=0).astype(jnp.bfloat16)
    bpre = (w0 - wc2 + wc4 + b1).reshape(1, TWO_H)
    W3a = W3[:HIDDEN]
    W3b = W3[HIDDEN:]
    posT = jnp.swapaxes(pos, 1, 2)

    full = lambda shape: pl.BlockSpec(shape, lambda b, i: (0,) * len(shape))
    grid = (B, N // TI)
    out = pl.pallas_call(
        _body,
        grid=grid,
        in_specs=[
            pl.BlockSpec((1, TI, HIDDEN), lambda b, i: (b, i, 0)),
            pl.BlockSpec((1, N, HIDDEN), lambda b, i: (b, 0, 0)),
            pl.BlockSpec((1, TI, 2), lambda b, i: (b, i, 0)),
            pl.BlockSpec((1, 2, N), lambda b, i: (b, 0, 0)),
            full((HIDDEN, TWO_H)),
            full((HIDDEN, TWO_H)),
            full((16, TWO_H)),
            full((1, TWO_H)),
            full((TWO_H, HIDDEN)),
            full((1, HIDDEN)),
            full((HIDDEN, HIDDEN)),
            full((HIDDEN, HIDDEN)),
            full((1, HIDDEN)),
            full((HIDDEN, HIDDEN)),
            full((1, HIDDEN)),
            full((1, HIDDEN)),
            full((1, HIDDEN)),
        ],
        out_specs=pl.BlockSpec((1, TI, HIDDEN), lambda b, i: (b, i, 0)),
        out_shape=jax.ShapeDtypeStruct((B, N, HIDDEN), jnp.float32),
        compiler_params=pltpu.CompilerParams(
            dimension_semantics=("parallel", "parallel")),
    )(h, h, pos, posT, W1a, W1b, WF, bpre, W2,
      b2.reshape(1, -1), W3a, W3b, b3.reshape(1, -1), W4,
      b4.reshape(1, -1), gamma.reshape(1, -1), beta.reshape(1, -1))
    return out


# speedup vs baseline: 1.1858x; 1.1858x over previous
import jax
import jax.numpy as jnp
import numpy as np
from jax.experimental import pallas as pl
from jax.experimental.pallas import tpu as pltpu

B = 4
N = 512
HIDDEN = 64
NUM_RADIAL = 8
MAX_ANG = 4
CUTOFF = 5.0
TWO_H = 2 * HIDDEN

TI = 64


def _body(hi_ref, hall_ref, posi_ref, posallT_ref,
          w1a_ref, w1b_ref, wf_ref, bpre_ref, w2_ref, b2_ref,
          w3a_ref, w3b_ref, b3_ref, w4_ref, b4_ref, g_ref, be_ref,
          o_ref):
    hi = hi_ref[0]
    hall = hall_ref[0]
    pi = posi_ref[0]
    pT = posallT_ref[0]

    pxi = pi[:, 0:1]
    pyi = pi[:, 1:2]
    pxj = pT[0:1, :]
    pyj = pT[1:2, :]

    pdx = pxi - pxj
    pdy = pyi - pyj
    d2 = pdx * pdx + pdy * pdy
    dist = jnp.sqrt(jnp.maximum(d2, 1e-16))
    inv = 1.0 / (dist + 1e-8)
    dx = pdx * inv
    dy = pdy * inv
    denom = dx * dx + dy * dy
    msk = denom < 1e-12
    xs = jnp.where(msk, 1.0, dx)
    ys = jnp.where(msk, 0.0, dy)
    rinv = jax.lax.rsqrt(jnp.where(msk, 1.0, denom))
    c1 = xs * rinv
    s1 = ys * rinv

    xq = dist * (1.0 / CUTOFF)
    x2 = xq * xq
    x6 = x2 * x2 * x2
    t1 = 1.0 - x6
    t2 = t1 * t1
    cut = jnp.where(xq < 1.0, t2 * t2 * t2, 0.0)
    csum = jnp.sum(cut, axis=1, keepdims=True)

    width_inv = NUM_RADIAL / CUTOFF
    dw = dist * width_inv
    cstep = NUM_RADIAL / (NUM_RADIAL - 1.0)
    planes = []
    for k in range(NUM_RADIAL):
        t = dw - (k * cstep)
        planes.append(jnp.exp(-(t * t)))
    cc2 = c1 * c1
    cc3 = cc2 * c1
    cc4 = cc2 * cc2
    planes += [c1, cc2, cc3, cc4, s1, s1 * c1, s1 * cc2, s1 * cc3]
    G = jnp.stack(planes, axis=0)

    rowA = (jnp.dot(hi, w1a_ref[...], preferred_element_type=jnp.float32)
            + bpre_ref[...]).astype(jnp.bfloat16)
    colB = jnp.dot(hall, w1b_ref[...],
                   preferred_element_type=jnp.float32).astype(jnp.bfloat16)
    pref = jax.lax.dot_general(
        G.astype(jnp.bfloat16), wf_ref[...], (((0,), (0,)), ((), ())),
        preferred_element_type=jnp.float32)
    pre = pref.astype(jnp.bfloat16) + rowA[:, None, :] + colB[None, :, :]
    half = jnp.bfloat16(0.5)
    hid = pre * (half * jnp.tanh(half * pre) + half)

    ch = jax.lax.dot_general(
        cut.astype(jnp.bfloat16), hid, (((1,), (1,)), ((0,), (0,))),
        preferred_element_type=jnp.float32)
    agg = (jnp.dot(ch, w2_ref[...], preferred_element_type=jnp.float32)
           + csum * b2_ref[...])

    ui = (jnp.dot(hi, w3a_ref[...], preferred_element_type=jnp.float32)
          + jnp.dot(agg, w3b_ref[...], preferred_element_type=jnp.float32)
          + b3_ref[...])
    u = ui * jax.nn.sigmoid(ui)
    u2 = jnp.dot(u, w4_ref[...], preferred_element_type=jnp.float32) + b4_ref[...]
    y = hi + u2
    mu = jnp.mean(y, axis=-1, keepdims=True)
    r = y - mu
    var = jnp.mean(r * r, axis=-1, keepdims=True)
    o_ref[0] = (r * jax.lax.rsqrt(var + 1e-5)) * g_ref[...] + be_ref[...]


@jax.jit
def kernel(h, pos, W1, b1, W2, b2, W3, b3, W4, b4, gamma, beta):
    W1a = W1[:HIDDEN]
    W1b = W1[HIDDEN:TWO_H]
    W1r = W1[TWO_H:TWO_H + NUM_RADIAL]
    Wang = W1[TWO_H + NUM_RADIAL:]
    w0, wc1, ws1, wc2, ws2, wc3, ws3, wc4, ws4 = [Wang[i] for i in range(9)]
    WF = jnp.concatenate([
        W1r,
        jnp.stack([
            wc1 - 3.0 * wc3,
            2.0 * wc2 - 8.0 * wc4,
            4.0 * wc3,
            8.0 * wc4,
            ws1 - ws3,
            2.0 * ws2 - 4.0 * ws4,
            4.0 * ws3,
            8.0 * ws4,
        ]),
    ], axis=0).astype(jnp.bfloat16)
    bpre = (w0 - wc2 + wc4 + b1).reshape(1, TWO_H)
    W3a = W3[:HIDDEN]
    W3b = W3[HIDDEN:]
    posT = jnp.swapaxes(pos, 1, 2)

    full = lambda shape: pl.BlockSpec(shape, lambda b, i: (0,) * len(shape))
    grid = (B, N // TI)
    out = pl.pallas_call(
        _body,
        grid=grid,
        in_specs=[
            pl.BlockSpec((1, TI, HIDDEN), lambda b, i: (b, i, 0)),
            pl.BlockSpec((1, N, HIDDEN), lambda b, i: (b, 0, 0)),
            pl.BlockSpec((1, TI, 2), lambda b, i: (b, i, 0)),
            pl.BlockSpec((1, 2, N), lambda b, i: (b, 0, 0)),
            full((HIDDEN, TWO_H)),
            full((HIDDEN, TWO_H)),
            full((16, TWO_H)),
            full((1, TWO_H)),
            full((TWO_H, HIDDEN)),
            full((1, HIDDEN)),
            full((HIDDEN, HIDDEN)),
            full((HIDDEN, HIDDEN)),
            full((1, HIDDEN)),
            full((HIDDEN, HIDDEN)),
            full((1, HIDDEN)),
            full((1, HIDDEN)),
            full((1, HIDDEN)),
        ],
        out_specs=pl.BlockSpec((1, TI, HIDDEN), lambda b, i: (b, i, 0)),
        out_shape=jax.ShapeDtypeStruct((B, N, HIDDEN), jnp.float32),
        compiler_params=pltpu.CompilerParams(
            dimension_semantics=("parallel", "parallel")),
    )(h, h, pos, posT, W1a, W1b, WF, bpre, W2,
      b2.reshape(1, -1), W3a, W3b, b3.reshape(1, -1), W4,
      b4.reshape(1, -1), gamma.reshape(1, -1), beta.reshape(1, -1))
    return out


# speedup vs baseline: 1.2115x; 1.0216x over previous
import jax
import jax.numpy as jnp
import numpy as np
from jax.experimental import pallas as pl
from jax.experimental.pallas import tpu as pltpu

B = 4
N = 512
HIDDEN = 64
NUM_RADIAL = 8
MAX_ANG = 4
CUTOFF = 5.0
TWO_H = 2 * HIDDEN

TI = 64


def _body(hi_ref, hall_ref, posi_ref, posallT_ref,
          w1a_ref, w1b_ref, wf_ref, bpre_ref, w2_ref, b2_ref,
          w3a_ref, w3b_ref, b3_ref, w4_ref, b4_ref, g_ref, be_ref,
          o_ref):
    hi = hi_ref[0]
    hall = hall_ref[0]
    pi = posi_ref[0]
    pT = posallT_ref[0]

    pxi = pi[:, 0:1]
    pyi = pi[:, 1:2]
    pxj = pT[0:1, :]
    pyj = pT[1:2, :]

    pdx = pxi - pxj
    pdy = pyi - pyj
    d2 = pdx * pdx + pdy * pdy
    dist = jnp.sqrt(jnp.maximum(d2, 1e-16))
    inv = 1.0 / (dist + 1e-8)
    dx = pdx * inv
    dy = pdy * inv
    denom = dx * dx + dy * dy
    msk = denom < 1e-12
    xs = jnp.where(msk, 1.0, dx)
    ys = jnp.where(msk, 0.0, dy)
    rinv = jax.lax.rsqrt(jnp.where(msk, 1.0, denom))
    c1 = xs * rinv
    s1 = ys * rinv

    xq = dist * (1.0 / CUTOFF)
    x2 = xq * xq
    x6 = x2 * x2 * x2
    t1 = 1.0 - x6
    t2 = t1 * t1
    cut = jnp.where(xq < 1.0, t2 * t2 * t2, 0.0)
    csum = jnp.sum(cut, axis=1, keepdims=True)

    width_inv = NUM_RADIAL / CUTOFF
    dw = dist * width_inv
    cstep = NUM_RADIAL / (NUM_RADIAL - 1.0)
    planes = []
    for k in range(NUM_RADIAL):
        t = dw - (k * cstep)
        planes.append(jnp.exp(-(t * t)))
    cc2 = c1 * c1
    cc3 = cc2 * c1
    cc4 = cc2 * cc2
    planes += [c1, cc2, cc3, cc4, s1, s1 * c1, s1 * cc2, s1 * cc3]
    G = jnp.stack(planes, axis=0)

    rowA = (jnp.dot(hi, w1a_ref[...], preferred_element_type=jnp.float32)
            + bpre_ref[...]).astype(jnp.bfloat16)
    colB = jnp.dot(hall, w1b_ref[...],
                   preferred_element_type=jnp.float32).astype(jnp.bfloat16)
    pref = jax.lax.dot_general(
        G.astype(jnp.bfloat16), wf_ref[...], (((0,), (0,)), ((), ())),
        preferred_element_type=jnp.float32)
    a = pref.astype(jnp.bfloat16) + rowA[:, None, :] + colB[None, :, :]
    hid = a + a * jnp.tanh(a)

    ch = jax.lax.dot_general(
        cut.astype(jnp.bfloat16), hid, (((1,), (1,)), ((0,), (0,))),
        preferred_element_type=jnp.float32)
    agg = (jnp.dot(ch, w2_ref[...], preferred_element_type=jnp.float32)
           + csum * b2_ref[...])

    ui = (jnp.dot(hi, w3a_ref[...], preferred_element_type=jnp.float32)
          + jnp.dot(agg, w3b_ref[...], preferred_element_type=jnp.float32)
          + b3_ref[...])
    u = ui * jax.nn.sigmoid(ui)
    u2 = jnp.dot(u, w4_ref[...], preferred_element_type=jnp.float32) + b4_ref[...]
    y = hi + u2
    mu = jnp.mean(y, axis=-1, keepdims=True)
    r = y - mu
    var = jnp.mean(r * r, axis=-1, keepdims=True)
    o_ref[0] = (r * jax.lax.rsqrt(var + 1e-5)) * g_ref[...] + be_ref[...]


@jax.jit
def kernel(h, pos, W1, b1, W2, b2, W3, b3, W4, b4, gamma, beta):
    W1a = W1[:HIDDEN]
    W1b = W1[HIDDEN:TWO_H]
    W1r = W1[TWO_H:TWO_H + NUM_RADIAL]
    Wang = W1[TWO_H + NUM_RADIAL:]
    w0, wc1, ws1, wc2, ws2, wc3, ws3, wc4, ws4 = [Wang[i] for i in range(9)]
    WF = (0.5 * jnp.concatenate([
        W1r,
        jnp.stack([
            wc1 - 3.0 * wc3,
            2.0 * wc2 - 8.0 * wc4,
            4.0 * wc3,
            8.0 * wc4,
            ws1 - ws3,
            2.0 * ws2 - 4.0 * ws4,
            4.0 * ws3,
            8.0 * ws4,
        ]),
    ], axis=0)).astype(jnp.bfloat16)
    bpre = (0.5 * (w0 - wc2 + wc4 + b1)).reshape(1, TWO_H)
    W1a = 0.5 * W1a
    W1b = 0.5 * W1b
    W3a = W3[:HIDDEN]
    W3b = W3[HIDDEN:]
    posT = jnp.swapaxes(pos, 1, 2)

    full = lambda shape: pl.BlockSpec(shape, lambda b, i: (0,) * len(shape))
    grid = (B, N // TI)
    out = pl.pallas_call(
        _body,
        grid=grid,
        in_specs=[
            pl.BlockSpec((1, TI, HIDDEN), lambda b, i: (b, i, 0)),
            pl.BlockSpec((1, N, HIDDEN), lambda b, i: (b, 0, 0)),
            pl.BlockSpec((1, TI, 2), lambda b, i: (b, i, 0)),
            pl.BlockSpec((1, 2, N), lambda b, i: (b, 0, 0)),
            full((HIDDEN, TWO_H)),
            full((HIDDEN, TWO_H)),
            full((16, TWO_H)),
            full((1, TWO_H)),
            full((TWO_H, HIDDEN)),
            full((1, HIDDEN)),
            full((HIDDEN, HIDDEN)),
            full((HIDDEN, HIDDEN)),
            full((1, HIDDEN)),
            full((HIDDEN, HIDDEN)),
            full((1, HIDDEN)),
            full((1, HIDDEN)),
            full((1, HIDDEN)),
        ],
        out_specs=pl.BlockSpec((1, TI, HIDDEN), lambda b, i: (b, i, 0)),
        out_shape=jax.ShapeDtypeStruct((B, N, HIDDEN), jnp.float32),
        compiler_params=pltpu.CompilerParams(
            dimension_semantics=("parallel", "parallel")),
    )(h, h, pos, posT, W1a, W1b, WF, bpre, W2,
      b2.reshape(1, -1), W3a, W3b, b3.reshape(1, -1), W4,
      b4.reshape(1, -1), gamma.reshape(1, -1), beta.reshape(1, -1))
    return out


# speedup vs baseline: 1.3026x; 1.0752x over previous
import jax
import jax.numpy as jnp
import numpy as np
from jax.experimental import pallas as pl
from jax.experimental.pallas import tpu as pltpu

B = 4
N = 512
HIDDEN = 64
NUM_RADIAL = 8
MAX_ANG = 4
CUTOFF = 5.0
TWO_H = 2 * HIDDEN

TI = 64


def _body(hi_ref, hall_ref, posi_ref, posallT_ref,
          w1a_ref, w1b_ref, wf_ref, bpre_ref, w2_ref, b2_ref,
          o_ref):
    hi = hi_ref[0]
    hall = hall_ref[0]
    pi = posi_ref[0]
    pT = posallT_ref[0]

    pxi = pi[:, 0:1]
    pyi = pi[:, 1:2]
    pxj = pT[0:1, :]
    pyj = pT[1:2, :]

    pdx = pxi - pxj
    pdy = pyi - pyj
    d2 = pdx * pdx + pdy * pdy
    dist = jnp.sqrt(jnp.maximum(d2, 1e-16))
    inv = 1.0 / (dist + 1e-8)
    dx = pdx * inv
    dy = pdy * inv
    denom = dx * dx + dy * dy
    msk = denom < 1e-12
    xs = jnp.where(msk, 1.0, dx)
    ys = jnp.where(msk, 0.0, dy)
    rinv = jax.lax.rsqrt(jnp.where(msk, 1.0, denom))
    c1 = xs * rinv
    s1 = ys * rinv

    xq = dist * (1.0 / CUTOFF)
    x2 = xq * xq
    x6 = x2 * x2 * x2
    t1 = 1.0 - x6
    t2 = t1 * t1
    cut = jnp.where(xq < 1.0, t2 * t2 * t2, 0.0)
    csum = jnp.sum(cut, axis=1, keepdims=True)

    width_inv = NUM_RADIAL / CUTOFF
    dw = dist * width_inv
    cstep = NUM_RADIAL / (NUM_RADIAL - 1.0)
    planes = []
    for k in range(NUM_RADIAL):
        t = dw - (k * cstep)
        planes.append(jnp.exp(-(t * t)))
    cc2 = c1 * c1
    cc3 = cc2 * c1
    cc4 = cc2 * cc2
    planes += [c1, cc2, cc3, cc4, s1, s1 * c1, s1 * cc2, s1 * cc3]
    G = jnp.stack(planes, axis=0)

    rowA = (jnp.dot(hi, w1a_ref[...], preferred_element_type=jnp.float32)
            + bpre_ref[...]).astype(jnp.bfloat16)
    colB = jnp.dot(hall, w1b_ref[...],
                   preferred_element_type=jnp.float32).astype(jnp.bfloat16)
    pref = jax.lax.dot_general(
        G.astype(jnp.bfloat16), wf_ref[...], (((0,), (0,)), ((), ())),
        preferred_element_type=jnp.float32)
    a = pref.astype(jnp.bfloat16) + rowA[:, None, :] + colB[None, :, :]
    hid = a + a * jnp.tanh(a)

    ch = jax.lax.dot_general(
        cut.astype(jnp.bfloat16), hid, (((1,), (1,)), ((0,), (0,))),
        preferred_element_type=jnp.float32)
    o_ref[0] = (jnp.dot(ch, w2_ref[...], preferred_element_type=jnp.float32)
                + csum * b2_ref[...])


def _body2(h_ref, agg_ref, w3a_ref, w3b_ref, b3_ref, w4_ref, b4_ref,
           g_ref, be_ref, o_ref):
    hi = h_ref[...]
    agg = agg_ref[...]
    ui = (jnp.dot(hi, w3a_ref[...], preferred_element_type=jnp.float32)
          + jnp.dot(agg, w3b_ref[...], preferred_element_type=jnp.float32)
          + b3_ref[...])
    u = ui * jax.nn.sigmoid(ui)
    u2 = jnp.dot(u, w4_ref[...], preferred_element_type=jnp.float32) + b4_ref[...]
    y = hi + u2
    mu = jnp.mean(y, axis=-1, keepdims=True)
    r = y - mu
    var = jnp.mean(r * r, axis=-1, keepdims=True)
    o_ref[...] = (r * jax.lax.rsqrt(var + 1e-5)) * g_ref[...] + be_ref[...]


@jax.jit
def kernel(h, pos, W1, b1, W2, b2, W3, b3, W4, b4, gamma, beta):
    W1a = W1[:HIDDEN]
    W1b = W1[HIDDEN:TWO_H]
    W1r = W1[TWO_H:TWO_H + NUM_RADIAL]
    Wang = W1[TWO_H + NUM_RADIAL:]
    w0, wc1, ws1, wc2, ws2, wc3, ws3, wc4, ws4 = [Wang[i] for i in range(9)]
    WF = (0.5 * jnp.concatenate([
        W1r,
        jnp.stack([
            wc1 - 3.0 * wc3,
            2.0 * wc2 - 8.0 * wc4,
            4.0 * wc3,
            8.0 * wc4,
            ws1 - ws3,
            2.0 * ws2 - 4.0 * ws4,
            4.0 * ws3,
            8.0 * ws4,
        ]),
    ], axis=0)).astype(jnp.bfloat16)
    bpre = (0.5 * (w0 - wc2 + wc4 + b1)).reshape(1, TWO_H)
    W1a = 0.5 * W1a
    W1b = 0.5 * W1b
    W3a = W3[:HIDDEN]
    W3b = W3[HIDDEN:]
    posT = jnp.swapaxes(pos, 1, 2)

    full = lambda shape: pl.BlockSpec(shape, lambda b, i: (0,) * len(shape))
    grid = (B, N // TI)
    agg = pl.pallas_call(
        _body,
        grid=grid,
        in_specs=[
            pl.BlockSpec((1, TI, HIDDEN), lambda b, i: (b, i, 0)),
            pl.BlockSpec((1, N, HIDDEN), lambda b, i: (b, 0, 0)),
            pl.BlockSpec((1, TI, 2), lambda b, i: (b, i, 0)),
            pl.BlockSpec((1, 2, N), lambda b, i: (b, 0, 0)),
            full((HIDDEN, TWO_H)),
            full((HIDDEN, TWO_H)),
            full((16, TWO_H)),
            full((1, TWO_H)),
            full((TWO_H, HIDDEN)),
            full((1, HIDDEN)),
        ],
        out_specs=pl.BlockSpec((1, TI, HIDDEN), lambda b, i: (b, i, 0)),
        out_shape=jax.ShapeDtypeStruct((B, N, HIDDEN), jnp.float32),
        compiler_params=pltpu.CompilerParams(
            dimension_semantics=("parallel", "parallel")),
    )(h, h, pos, posT, W1a, W1b, WF, bpre, W2, b2.reshape(1, -1))

    BN = B * N
    full2 = lambda shape: pl.BlockSpec(shape, lambda: (0,) * len(shape))
    out = pl.pallas_call(
        _body2,
        in_specs=[
            full2((BN, HIDDEN)),
            full2((BN, HIDDEN)),
            full2((HIDDEN, HIDDEN)),
            full2((HIDDEN, HIDDEN)),
            full2((1, HIDDEN)),
            full2((HIDDEN, HIDDEN)),
            full2((1, HIDDEN)),
            full2((1, HIDDEN)),
            full2((1, HIDDEN)),
        ],
        out_specs=full2((BN, HIDDEN)),
        out_shape=jax.ShapeDtypeStruct((BN, HIDDEN), jnp.float32),
    )(h.reshape(BN, HIDDEN), agg.reshape(BN, HIDDEN), W3a, W3b,
      b3.reshape(1, -1), W4, b4.reshape(1, -1),
      gamma.reshape(1, -1), beta.reshape(1, -1))
    return out.reshape(B, N, HIDDEN)


# speedup vs baseline: 1.3621x; 1.0456x over previous
import jax
import jax.numpy as jnp
import numpy as np
from jax.experimental import pallas as pl
from jax.experimental.pallas import tpu as pltpu

B = 4
N = 512
HIDDEN = 64
NUM_RADIAL = 8
MAX_ANG = 4
CUTOFF = 5.0
TWO_H = 2 * HIDDEN

TI = 128


def _body(hi_ref, hall_ref, posi_ref, posallT_ref,
          w1a_ref, w1b_ref, wf_ref, bpre_ref, w2_ref, b2_ref,
          o_ref):
    hi = hi_ref[0]
    hall = hall_ref[0]
    pi = posi_ref[0]
    pT = posallT_ref[0]

    pxi = pi[:, 0:1]
    pyi = pi[:, 1:2]
    pxj = pT[0:1, :]
    pyj = pT[1:2, :]

    pdx = pxi - pxj
    pdy = pyi - pyj
    d2 = pdx * pdx + pdy * pdy
    dist = jnp.sqrt(jnp.maximum(d2, 1e-16))
    inv = 1.0 / (dist + 1e-8)
    dx = pdx * inv
    dy = pdy * inv
    denom = dx * dx + dy * dy
    msk = denom < 1e-12
    xs = jnp.where(msk, 1.0, dx)
    ys = jnp.where(msk, 0.0, dy)
    rinv = jax.lax.rsqrt(jnp.where(msk, 1.0, denom))
    c1 = xs * rinv
    s1 = ys * rinv

    xq = dist * (1.0 / CUTOFF)
    x2 = xq * xq
    x6 = x2 * x2 * x2
    t1 = 1.0 - x6
    t2 = t1 * t1
    cut = jnp.where(xq < 1.0, t2 * t2 * t2, 0.0)
    csum = jnp.sum(cut, axis=1, keepdims=True)

    width_inv = NUM_RADIAL / CUTOFF
    dw = dist * width_inv
    cstep = NUM_RADIAL / (NUM_RADIAL - 1.0)
    planes = []
    for k in range(NUM_RADIAL):
        t = dw - (k * cstep)
        planes.append(jnp.exp(-(t * t)))
    cc2 = c1 * c1
    cc3 = cc2 * c1
    cc4 = cc2 * cc2
    planes += [c1, cc2, cc3, cc4, s1, s1 * c1, s1 * cc2, s1 * cc3]
    G = jnp.stack(planes, axis=0)

    rowA = (jnp.dot(hi, w1a_ref[...], preferred_element_type=jnp.float32)
            + bpre_ref[...]).astype(jnp.bfloat16)
    colB = jnp.dot(hall, w1b_ref[...],
                   preferred_element_type=jnp.float32).astype(jnp.bfloat16)
    pref = jax.lax.dot_general(
        G.astype(jnp.bfloat16), wf_ref[...], (((0,), (0,)), ((), ())),
        preferred_element_type=jnp.float32)
    a = pref.astype(jnp.bfloat16) + rowA[:, None, :] + colB[None, :, :]
    hid = a + a * jnp.tanh(a)

    ch = jax.lax.dot_general(
        cut.astype(jnp.bfloat16), hid, (((1,), (1,)), ((0,), (0,))),
        preferred_element_type=jnp.float32)
    o_ref[0] = (jnp.dot(ch, w2_ref[...], preferred_element_type=jnp.float32)
                + csum * b2_ref[...])


def _body2(h_ref, agg_ref, w3a_ref, w3b_ref, b3_ref, w4_ref, b4_ref,
           g_ref, be_ref, o_ref):
    hi = h_ref[...]
    agg = agg_ref[...]
    ui = (jnp.dot(hi, w3a_ref[...], preferred_element_type=jnp.float32)
          + jnp.dot(agg, w3b_ref[...], preferred_element_type=jnp.float32)
          + b3_ref[...])
    u = ui * jax.nn.sigmoid(ui)
    u2 = jnp.dot(u, w4_ref[...], preferred_element_type=jnp.float32) + b4_ref[...]
    y = hi + u2
    mu = jnp.mean(y, axis=-1, keepdims=True)
    r = y - mu
    var = jnp.mean(r * r, axis=-1, keepdims=True)
    o_ref[...] = (r * jax.lax.rsqrt(var + 1e-5)) * g_ref[...] + be_ref[...]


@jax.jit
def kernel(h, pos, W1, b1, W2, b2, W3, b3, W4, b4, gamma, beta):
    W1a = W1[:HIDDEN]
    W1b = W1[HIDDEN:TWO_H]
    W1r = W1[TWO_H:TWO_H + NUM_RADIAL]
    Wang = W1[TWO_H + NUM_RADIAL:]
    w0, wc1, ws1, wc2, ws2, wc3, ws3, wc4, ws4 = [Wang[i] for i in range(9)]
    WF = (0.5 * jnp.concatenate([
        W1r,
        jnp.stack([
            wc1 - 3.0 * wc3,
            2.0 * wc2 - 8.0 * wc4,
            4.0 * wc3,
            8.0 * wc4,
            ws1 - ws3,
            2.0 * ws2 - 4.0 * ws4,
            4.0 * ws3,
            8.0 * ws4,
        ]),
    ], axis=0)).astype(jnp.bfloat16)
    bpre = (0.5 * (w0 - wc2 + wc4 + b1)).reshape(1, TWO_H)
    W1a = 0.5 * W1a
    W1b = 0.5 * W1b
    W3a = W3[:HIDDEN]
    W3b = W3[HIDDEN:]
    posT = jnp.swapaxes(pos, 1, 2)

    full = lambda shape: pl.BlockSpec(shape, lambda b, i: (0,) * len(shape))
    grid = (B, N // TI)
    agg = pl.pallas_call(
        _body,
        grid=grid,
        in_specs=[
            pl.BlockSpec((1, TI, HIDDEN), lambda b, i: (b, i, 0)),
            pl.BlockSpec((1, N, HIDDEN), lambda b, i: (b, 0, 0)),
            pl.BlockSpec((1, TI, 2), lambda b, i: (b, i, 0)),
            pl.BlockSpec((1, 2, N), lambda b, i: (b, 0, 0)),
            full((HIDDEN, TWO_H)),
            full((HIDDEN, TWO_H)),
            full((16, TWO_H)),
            full((1, TWO_H)),
            full((TWO_H, HIDDEN)),
            full((1, HIDDEN)),
        ],
        out_specs=pl.BlockSpec((1, TI, HIDDEN), lambda b, i: (b, i, 0)),
        out_shape=jax.ShapeDtypeStruct((B, N, HIDDEN), jnp.float32),
        compiler_params=pltpu.CompilerParams(
            dimension_semantics=("parallel", "parallel")),
    )(h, h, pos, posT, W1a, W1b, WF, bpre, W2, b2.reshape(1, -1))

    BN = B * N
    full2 = lambda shape: pl.BlockSpec(shape, lambda: (0,) * len(shape))
    out = pl.pallas_call(
        _body2,
        in_specs=[
            full2((BN, HIDDEN)),
            full2((BN, HIDDEN)),
            full2((HIDDEN, HIDDEN)),
            full2((HIDDEN, HIDDEN)),
            full2((1, HIDDEN)),
            full2((HIDDEN, HIDDEN)),
            full2((1, HIDDEN)),
            full2((1, HIDDEN)),
            full2((1, HIDDEN)),
        ],
        out_specs=full2((BN, HIDDEN)),
        out_shape=jax.ShapeDtypeStruct((BN, HIDDEN), jnp.float32),
    )(h.reshape(BN, HIDDEN), agg.reshape(BN, HIDDEN), W3a, W3b,
      b3.reshape(1, -1), W4, b4.reshape(1, -1),
      gamma.reshape(1, -1), beta.reshape(1, -1))
    return out.reshape(B, N, HIDDEN)
